# baseline (speedup 1.0000x reference)
import functools

import jax
import jax.numpy as jnp
from jax import lax
from jax.experimental import pallas as pl
from jax.experimental.pallas import tpu as pltpu
from jax.experimental.pallas import tpu_sc as plsc

B = 16384
D_BATCH, D_DONOR, D_ASSAY = 64, 64, 32
D_OUT = D_BATCH + D_DONOR + D_ASSAY

NC, NS = 2, 16
NW = NC * NS
BPW = B // NW

_mesh = plsc.VectorSubcoreMesh(core_axis_name="c", subcore_axis_name="s")


@functools.partial(
    pl.kernel,
    out_type=jax.ShapeDtypeStruct((B, D_OUT), jnp.float32),
    mesh=_mesh,
    scratch_types=[
        pltpu.VMEM((BPW,), jnp.int32),
        pltpu.VMEM((BPW,), jnp.int32),
        pltpu.VMEM((BPW,), jnp.int32),
        pltpu.VMEM((BPW, D_BATCH), jnp.float32),
        pltpu.VMEM((BPW, D_DONOR), jnp.float32),
        pltpu.VMEM((BPW, D_ASSAY), jnp.float32),
        pltpu.SemaphoreType.DMA,
    ],
    compiler_params=pltpu.CompilerParams(use_tc_tiling_on_sc=False),
)
def _embed_concat(b_idx, d_idx, a_idx, Wb, Wd, Wa, out,
                  ib, idn, ia, rb, rd, ra, sem):
    wid = lax.axis_index("c") * NS + lax.axis_index("s")
    base = wid * BPW
    rows = pl.ds(base, BPW)
    iload = [
        pltpu.async_copy(b_idx.at[rows], ib, sem),
        pltpu.async_copy(d_idx.at[rows], idn, sem),
        pltpu.async_copy(a_idx.at[rows], ia, sem),
    ]
    for c in iload:
        c.wait()
    gathers = [
        pltpu.async_copy(Wb.at[ib], rb, sem),
        pltpu.async_copy(Wd.at[idn], rd, sem),
        pltpu.async_copy(Wa.at[ia], ra, sem),
    ]
    for c in gathers:
        c.wait()
    pltpu.sync_copy(rb, out.at[rows, pl.ds(0, D_BATCH)])
    pltpu.sync_copy(rd, out.at[rows, pl.ds(D_BATCH, D_DONOR)])
    pltpu.sync_copy(ra, out.at[rows, pl.ds(D_BATCH + D_DONOR, D_ASSAY)])


def kernel(batch, donor, assay, W_batch, W_donor, W_assay):
    b1 = batch.astype(jnp.int32)
    d1 = donor.astype(jnp.int32)
    a1 = assay.astype(jnp.int32)
    return _embed_concat(b1, d1, a1, W_batch, W_donor, W_assay)
